# baseline (speedup 1.0000x reference)
import functools

import jax
import jax.numpy as jnp
from jax import lax
from jax.experimental import pallas as pl
from jax.experimental.pallas import tpu as pltpu
from jax.experimental.pallas import tpu_sc as plsc

N_NODES = 10000
N_PAD = 10240
FEAT = 128
N_EDGES = 320000
NC = 2
NS = 16
NW = NC * NS
CHUNK = 64
NCHUNKS = 160
EPW = NCHUNKS * CHUNK
E_PAD = NW * EPW
RPT = N_PAD // NS
NBUF = 2


def _fill(ref, val, nrows, ncols):
    val16 = jnp.full((16,), val, jnp.float32)
    if ncols is None:
        def body(i, _):
            ref[pl.ds(i * 16, 16)] = val16
            return 0
        lax.fori_loop(0, nrows // 16, body, 0)
    else:
        def row(i, _):
            def col(j, _):
                ref[i, pl.ds(j * 16, 16)] = val16
                return 0
            return lax.fori_loop(0, ncols // 16, col, 0)
        lax.fori_loop(0, nrows, row, 0)


def _make_agg(with_cnt: bool):
    mesh = plsc.VectorSubcoreMesh(core_axis_name="c", subcore_axis_name="s")
    out_type = [jax.ShapeDtypeStruct((NC, N_PAD, FEAT), jnp.float32)]
    scratch = [
        pltpu.VMEM((NCHUNKS, CHUNK), jnp.int32),
        pltpu.VMEM((NBUF, CHUNK), jnp.int32),
        pltpu.VMEM((NBUF, CHUNK, FEAT), jnp.float32),
        pltpu.VMEM_SHARED((N_PAD, FEAT), jnp.float32),
        pltpu.SemaphoreType.DMA((NBUF,)),
        pltpu.SemaphoreType.DMA((NBUF,)),
        pltpu.SemaphoreType.DMA((NBUF,)),
    ]
    if with_cnt:
        out_type.append(jax.ShapeDtypeStruct((NC, N_PAD), jnp.float32))
        scratch += [
            pltpu.VMEM((CHUNK,), jnp.float32),
            pltpu.VMEM((RPT,), jnp.float32),
            pltpu.VMEM_SHARED((N_PAD,), jnp.float32),
            pltpu.SemaphoreType.DMA((NBUF,)),
        ]

    def body(feat, src, dst, *rest):
        if with_cnt:
            (out_sum, out_cnt, didx, sidx, rows, ssum, isem, gsem, ssem,
             ones, zc, scnt, csem) = rest
        else:
            out_sum, didx, sidx, rows, ssum, isem, gsem, ssem = rest
        cid = lax.axis_index("c")
        sid = lax.axis_index("s")
        wid = cid * NS + sid
        rb = sid * RPT

        zbuf = rows.at[0]
        _fill(zbuf, 0.0, CHUNK, FEAT)
        for k in range(RPT // CHUNK):
            pltpu.sync_copy(zbuf, ssum.at[pl.ds(rb + k * CHUNK, CHUNK), :])
        if with_cnt:
            _fill(ones, 1.0, CHUNK, None)
            _fill(zc, 0.0, RPT, None)
            pltpu.sync_copy(zc, scnt.at[pl.ds(rb, RPT)])

        pltpu.sync_copy(dst.at[wid], didx)
        plsc.subcore_barrier()

        def idx_load(i, p):
            pltpu.async_copy(src.at[wid, i], sidx.at[p], isem.at[p])

        def idx_wait(i, p):
            pltpu.make_async_copy(src.at[wid, i], sidx.at[p],
                                  isem.at[p]).wait()

        def gather(i, p):
            pltpu.async_copy(feat.at[sidx.at[p]], rows.at[p], gsem.at[p])

        def gather_wait(i, p):
            pltpu.make_async_copy(feat.at[sidx.at[p]], rows.at[p],
                                  gsem.at[p]).wait()

        def scatter(i, p):
            pltpu.async_copy(rows.at[p], ssum.at[didx.at[i]], ssem.at[p],
                             add=True)
            if with_cnt:
                pltpu.async_copy(ones, scnt.at[didx.at[i]], csem.at[p],
                                 add=True)

        def scatter_wait(i, p):
            pltpu.make_async_copy(rows.at[p], ssum.at[didx.at[i]],
                                  ssem.at[p]).wait()
            if with_cnt:
                pltpu.make_async_copy(ones, scnt.at[didx.at[i]],
                                      csem.at[p]).wait()

        idx_load(0, 0)
        idx_load(1, 1)
        idx_wait(0, 0)
        gather(0, 0)

        def chunk(i, _):
            p = lax.rem(i, NBUF)
            q = 1 - p

            @pl.when(i >= 1)
            def _():
                scatter_wait(i - 1, q)

            @pl.when(i + 1 < NCHUNKS)
            def _():
                idx_wait(i + 1, q)
                gather(i + 1, q)

            gather_wait(i, p)

            @pl.when(i + 2 < NCHUNKS)
            def _():
                idx_load(i + 2, p)

            scatter(i, p)
            return 0

        lax.fori_loop(0, NCHUNKS, chunk, 0)
        scatter_wait(NCHUNKS - 1, lax.rem(NCHUNKS - 1, NBUF))
        plsc.subcore_barrier()

        pltpu.sync_copy(ssum.at[pl.ds(rb, RPT), :],
                        out_sum.at[cid, pl.ds(rb, RPT), :])
        if with_cnt:
            pltpu.sync_copy(scnt.at[pl.ds(rb, RPT)],
                            out_cnt.at[cid, pl.ds(rb, RPT)])

    return pl.kernel(body, out_type=out_type, mesh=mesh,
                     scratch_types=scratch)


_agg_cnt = _make_agg(True)
_agg = _make_agg(False)

BR = 2048


def _dense_body(relu):
    def body(sp_ref, cp_ref, x_ref, wl_ref, wr_ref, b_ref, o_ref):
        c = jnp.clip(cp_ref[0] + cp_ref[1], 1.0, None)
        mean = (sp_ref[0] + sp_ref[1]) / c[:, None]
        acc = jnp.dot(mean, wl_ref[...], preferred_element_type=jnp.float32)
        acc = acc + jnp.dot(x_ref[...], wr_ref[...],
                            preferred_element_type=jnp.float32)
        acc = acc + b_ref[...]
        if relu:
            acc = jnp.maximum(acc, 0.0)
        o_ref[...] = acc
    return body


def _dense_layer(sp, cp, x, wl, wr, b, relu):
    return pl.pallas_call(
        _dense_body(relu),
        grid=(N_PAD // BR,),
        in_specs=[
            pl.BlockSpec((NC, BR, FEAT), lambda i: (0, i, 0)),
            pl.BlockSpec((NC, BR), lambda i: (0, i)),
            pl.BlockSpec((BR, FEAT), lambda i: (i, 0)),
            pl.BlockSpec((FEAT, FEAT), lambda i: (0, 0)),
            pl.BlockSpec((FEAT, FEAT), lambda i: (0, 0)),
            pl.BlockSpec((1, FEAT), lambda i: (0, 0)),
        ],
        out_specs=pl.BlockSpec((BR, FEAT), lambda i: (i, 0)),
        out_shape=jax.ShapeDtypeStruct((N_PAD, FEAT), jnp.float32),
    )(sp, cp, x, wl, wr, b)


def kernel(x, edge_index, Wl1, Wr1, b1, Wl2, Wr2, b2):
    ei = edge_index.astype(jnp.int32)
    pad = jnp.full((2, E_PAD - N_EDGES), N_NODES, jnp.int32)
    ei = jnp.concatenate([ei, pad], axis=1)
    src = ei[0].reshape(NW, NCHUNKS, CHUNK)
    dst = ei[1].reshape(NW, NCHUNKS, CHUNK)
    x_pad = jnp.pad(x, ((0, N_PAD - N_NODES), (0, 0)))
    sp1, cp = _agg_cnt(x_pad, src, dst)
    h = _dense_layer(sp1, cp, x_pad, Wl1, Wr1, b1.reshape(1, FEAT), True)
    sp2, = _agg(h, src, dst)
    out = _dense_layer(sp2, cp, h, Wl2, Wr2, b2.reshape(1, FEAT), False)
    return out[:N_NODES]


# speedup vs baseline: 2.2910x; 2.2910x over previous
import functools

import jax
import jax.numpy as jnp
from jax import lax
from jax.experimental import pallas as pl
from jax.experimental.pallas import tpu as pltpu
from jax.experimental.pallas import tpu_sc as plsc

N_NODES = 10000
N_PAD = 10240
FEAT = 128
N_EDGES = 320000
NC = 2
NS = 16
NW = NC * NS
CHUNK = 80
NCHUNKS = 125
EPW = NCHUNKS * CHUNK
E_PAD = NW * EPW
RPT = N_PAD // NS
ZCH = 64


def _fill(ref, val, nrows, ncols):
    val16 = jnp.full((16,), val, jnp.float32)
    if ncols is None:
        def body(i, _):
            ref[pl.ds(i * 16, 16)] = val16
            return 0
        lax.fori_loop(0, nrows // 16, body, 0)
    else:
        def row(i, _):
            def col(j, _):
                ref[i, pl.ds(j * 16, 16)] = val16
                return 0
            return lax.fori_loop(0, ncols // 16, col, 0)
        lax.fori_loop(0, nrows, row, 0)


def _make_agg(with_cnt: bool):
    mesh = plsc.VectorSubcoreMesh(core_axis_name="c", subcore_axis_name="s")
    out_type = [jax.ShapeDtypeStruct((NC, N_PAD, FEAT), jnp.float32)]
    scratch = [
        pltpu.VMEM((NCHUNKS, CHUNK), jnp.int32),
        pltpu.VMEM((NCHUNKS, CHUNK), jnp.int32),
        pltpu.VMEM((CHUNK, FEAT), jnp.float32),
        pltpu.VMEM_SHARED((N_PAD, FEAT), jnp.float32),
        pltpu.SemaphoreType.DMA,
    ]
    if with_cnt:
        out_type.append(jax.ShapeDtypeStruct((NC, N_PAD), jnp.float32))
        scratch += [
            pltpu.VMEM((CHUNK,), jnp.float32),
            pltpu.VMEM((RPT,), jnp.float32),
            pltpu.VMEM_SHARED((N_PAD,), jnp.float32),
        ]

    def body(feat, src, dst, *rest):
        if with_cnt:
            (out_sum, out_cnt, didx, sidx, rows, ssum, gsem,
             ones, zc, scnt) = rest
        else:
            out_sum, didx, sidx, rows, ssum, gsem = rest
        cid = lax.axis_index("c")
        sid = lax.axis_index("s")
        wid = cid * NS + sid
        rb = sid * RPT

        zbuf = rows.at[pl.ds(0, ZCH)]
        _fill(zbuf, 0.0, ZCH, FEAT)
        for k in range(RPT // ZCH):
            pltpu.sync_copy(zbuf, ssum.at[pl.ds(rb + k * ZCH, ZCH), :])
        if with_cnt:
            _fill(ones, 1.0, CHUNK, None)
            _fill(zc, 0.0, RPT, None)
            pltpu.sync_copy(zc, scnt.at[pl.ds(rb, RPT)])

        pltpu.sync_copy(dst.at[wid], didx)
        pltpu.sync_copy(src.at[wid], sidx)
        plsc.subcore_barrier()

        def chunk(i, _):
            pltpu.async_copy(feat.at[sidx.at[i]], rows, gsem).wait()
            pltpu.sync_copy(rows, ssum.at[didx.at[i]], add=True)
            if with_cnt:
                pltpu.sync_copy(ones, scnt.at[didx.at[i]], add=True)
            return 0

        lax.fori_loop(0, NCHUNKS, chunk, 0)
        plsc.subcore_barrier()

        pltpu.sync_copy(ssum.at[pl.ds(rb, RPT), :],
                        out_sum.at[cid, pl.ds(rb, RPT), :])
        if with_cnt:
            pltpu.sync_copy(scnt.at[pl.ds(rb, RPT)],
                            out_cnt.at[cid, pl.ds(rb, RPT)])

    return pl.kernel(body, out_type=out_type, mesh=mesh,
                     scratch_types=scratch)


_agg_cnt = _make_agg(True)
_agg = _make_agg(False)

BR = 2048


def _dense_body(relu):
    def body(sp_ref, cp_ref, x_ref, wl_ref, wr_ref, b_ref, o_ref):
        c = jnp.clip(cp_ref[0] + cp_ref[1], 1.0, None)
        mean = (sp_ref[0] + sp_ref[1]) / c[:, None]
        acc = jnp.dot(mean, wl_ref[...], preferred_element_type=jnp.float32)
        acc = acc + jnp.dot(x_ref[...], wr_ref[...],
                            preferred_element_type=jnp.float32)
        acc = acc + b_ref[...]
        if relu:
            acc = jnp.maximum(acc, 0.0)
        o_ref[...] = acc
    return body


def _dense_layer(sp, cp, x, wl, wr, b, relu):
    return pl.pallas_call(
        _dense_body(relu),
        grid=(N_PAD // BR,),
        in_specs=[
            pl.BlockSpec((NC, BR, FEAT), lambda i: (0, i, 0)),
            pl.BlockSpec((NC, BR), lambda i: (0, i)),
            pl.BlockSpec((BR, FEAT), lambda i: (i, 0)),
            pl.BlockSpec((FEAT, FEAT), lambda i: (0, 0)),
            pl.BlockSpec((FEAT, FEAT), lambda i: (0, 0)),
            pl.BlockSpec((1, FEAT), lambda i: (0, 0)),
        ],
        out_specs=pl.BlockSpec((BR, FEAT), lambda i: (i, 0)),
        out_shape=jax.ShapeDtypeStruct((N_PAD, FEAT), jnp.float32),
    )(sp, cp, x, wl, wr, b)


def kernel(x, edge_index, Wl1, Wr1, b1, Wl2, Wr2, b2):
    ei = edge_index.astype(jnp.int32)
    src = ei[0].reshape(NW, NCHUNKS, CHUNK)
    dst = ei[1].reshape(NW, NCHUNKS, CHUNK)
    x_pad = jnp.pad(x, ((0, N_PAD - N_NODES), (0, 0)))
    sp1, cp = _agg_cnt(x_pad, src, dst)
    h = _dense_layer(sp1, cp, x_pad, Wl1, Wr1, b1.reshape(1, FEAT), True)
    sp2, = _agg(h, src, dst)
    out = _dense_layer(sp2, cp, h, Wl2, Wr2, b2.reshape(1, FEAT), False)
    return out[:N_NODES]
